# Initial kernel scaffold; baseline (speedup 1.0000x reference)
import jax
import jax.numpy as jnp
from jax.experimental import pallas as pl


def kernel(xq, xb):
    raise NotImplementedError("write your pallas kernel here")


# speedup vs baseline: 2.1785x; 2.1785x over previous
import jax
import jax.numpy as jnp
from jax.experimental import pallas as pl

K_SEL = 21
Q = 1024
KRUN = 32
TILE = 256


def _stage(v, i, d, desc):
    n, t = v.shape
    vr = v.reshape(n // (2 * d), 2, d, t)
    ir = i.reshape(n // (2 * d), 2, d, t)
    sel = (vr[:, 0] >= vr[:, 1]) == desc
    fv = jnp.where(sel, vr[:, 0], vr[:, 1])
    fi = jnp.where(sel, ir[:, 0], ir[:, 1])
    sv = jnp.where(sel, vr[:, 1], vr[:, 0])
    si = jnp.where(sel, ir[:, 1], ir[:, 0])
    v = jnp.stack([fv, sv], axis=1).reshape(n, t)
    i = jnp.stack([fi, si], axis=1).reshape(n, t)
    return v, i


def _topk_body(xq_ref, xbt_ref, out_ref):
    s = jnp.dot(xq_ref[...], xbt_ref[...],
                preferred_element_type=jnp.float32)
    t = s.shape[-1]
    v = s
    i = jax.lax.broadcasted_iota(jnp.int32, (Q, t), 0)

    pos = i
    n = Q
    k = 2
    while k <= KRUN:
        d = k // 2
        while d >= 1:
            pr = pos.reshape(n // (2 * d), 2, d, t)[:, 0]
            even_run = (pr & KRUN) == 0
            if k == KRUN:
                desc = even_run
            else:
                desc = ((pr & k) == 0) == even_run
            v, i = _stage(v, i, d, desc)
            d //= 2
        k *= 2

    while v.shape[0] > KRUN:
        n = v.shape[0]
        vr = v.reshape(n // (2 * KRUN), 2, KRUN, t)
        ir = i.reshape(n // (2 * KRUN), 2, KRUN, t)
        m = vr[:, 0] >= vr[:, 1]
        v = jnp.where(m, vr[:, 0], vr[:, 1]).reshape(n // 2, t)
        i = jnp.where(m, ir[:, 0], ir[:, 1]).reshape(n // 2, t)
        n = v.shape[0]
        pos = jax.lax.broadcasted_iota(jnp.int32, (n, t), 0)
        d = KRUN // 2
        while d >= 1:
            pr = pos.reshape(n // (2 * d), 2, d, t)[:, 0]
            desc = (pr & KRUN) == 0
            v, i = _stage(v, i, d, desc)
            d //= 2

    for off in (0, 1):
        va = v[off:off + 30].reshape(15, 2, t)
        ia = i[off:off + 30].reshape(15, 2, t)
        beats = (va[:, 0] > va[:, 1]) | (
            (va[:, 0] == va[:, 1]) & (ia[:, 0] < ia[:, 1]))
        hv = jnp.where(beats, va[:, 0], va[:, 1])
        hi = jnp.where(beats, ia[:, 0], ia[:, 1])
        lv = jnp.where(beats, va[:, 1], va[:, 0])
        li = jnp.where(beats, ia[:, 1], ia[:, 0])
        vm = jnp.stack([hv, lv], axis=1).reshape(30, t)
        im = jnp.stack([hi, li], axis=1).reshape(30, t)
        vparts = [vm, v[off + 30:]] if off == 0 else [v[:off], vm, v[off + 30:]]
        iparts = [im, i[off + 30:]] if off == 0 else [i[:off], im, i[off + 30:]]
        v = jnp.concatenate(vparts, axis=0)
        i = jnp.concatenate(iparts, axis=0)

    out_ref[...] = i[:K_SEL]


def kernel(xq, xb):
    n = xb.shape[0]
    n_pad = ((n + TILE - 1) // TILE) * TILE
    xbt = jnp.pad(xb, ((0, n_pad - n), (0, 0))).T

    out = pl.pallas_call(
        _topk_body,
        grid=(n_pad // TILE,),
        in_specs=[
            pl.BlockSpec((Q, 16), lambda j: (0, 0)),
            pl.BlockSpec((16, TILE), lambda j: (0, j)),
        ],
        out_specs=pl.BlockSpec((K_SEL, TILE), lambda j: (0, j)),
        out_shape=jax.ShapeDtypeStruct((K_SEL, n_pad), jnp.int32),
    )(xq, xbt)
    return out[:, :n]


# speedup vs baseline: 16.3423x; 7.5016x over previous
import jax
import jax.numpy as jnp
from jax.experimental import pallas as pl

K_SEL = 21
Q = 1024
KRUN = 32
TILE = 256


def _stage(v, i, d, desc):
    n, t = v.shape
    vr = v.reshape(n // (2 * d), 2, d, t)
    ir = i.reshape(n // (2 * d), 2, d, t)
    sel = vr[:, 0] >= vr[:, 1]
    if desc is not True:
        sel = sel == desc
    fv = jnp.where(sel, vr[:, 0], vr[:, 1])
    fi = jnp.where(sel, ir[:, 0], ir[:, 1])
    sv = jnp.where(sel, vr[:, 1], vr[:, 0])
    si = jnp.where(sel, ir[:, 1], ir[:, 0])
    v = jnp.stack([fv, sv], axis=1).reshape(n, t)
    i = jnp.stack([fi, si], axis=1).reshape(n, t)
    return v, i


def _topk_body(xq_ref, xbt_ref, out_ref):
    s = jnp.dot(xq_ref[...], xbt_ref[...],
                preferred_element_type=jnp.float32)
    t = s.shape[-1]
    v = s
    i = jax.lax.broadcasted_iota(jnp.int32, (Q, t), 0)


    pos = i
    n = Q
    s = KRUN
    k = 2
    while k <= KRUN:
        d = k // 2
        while d >= 1:
            pr = pos.reshape(n // (2 * d * s), 2, d * s, t)[:, 0]
            desc = (((pr >> 5) & k) == 0) == ((pr & (s // 2)) == 0)
            v, i = _stage(v, i, d * s, desc)
            d //= 2
        k *= 2

    while s > 1:
        n = v.shape[0]
        vr = v.reshape(n // s, 2, s // 2, t)
        ir = i.reshape(n // s, 2, s // 2, t)
        m = vr[:, 0] >= vr[:, 1]
        v = jnp.where(m, vr[:, 0], vr[:, 1]).reshape(n // 2, t)
        i = jnp.where(m, ir[:, 0], ir[:, 1]).reshape(n // 2, t)
        s //= 2
        n = v.shape[0]
        pos = jax.lax.broadcasted_iota(jnp.int32, (n, t), 0)
        d = KRUN // 2
        while d >= 1:
            if s > 1:
                pr = pos.reshape(n // (2 * d * s), 2, d * s, t)[:, 0]
                desc = (pr & (s // 2)) == 0
            else:
                desc = True
            v, i = _stage(v, i, d * s, desc)
            d //= 2

    for off in (0, 1):
        va = v[off:off + 30].reshape(15, 2, t)
        ia = i[off:off + 30].reshape(15, 2, t)
        beats = (va[:, 0] > va[:, 1]) | (
            (va[:, 0] == va[:, 1]) & (ia[:, 0] < ia[:, 1]))
        hv = jnp.where(beats, va[:, 0], va[:, 1])
        hi = jnp.where(beats, ia[:, 0], ia[:, 1])
        lv = jnp.where(beats, va[:, 1], va[:, 0])
        li = jnp.where(beats, ia[:, 1], ia[:, 0])
        vm = jnp.stack([hv, lv], axis=1).reshape(30, t)
        im = jnp.stack([hi, li], axis=1).reshape(30, t)
        vparts = [vm, v[off + 30:]] if off == 0 else [v[:off], vm, v[off + 30:]]
        iparts = [im, i[off + 30:]] if off == 0 else [i[:off], im, i[off + 30:]]
        v = jnp.concatenate(vparts, axis=0)
        i = jnp.concatenate(iparts, axis=0)

    out_ref[...] = i[:K_SEL]


def kernel(xq, xb):
    n = xb.shape[0]
    n_pad = ((n + TILE - 1) // TILE) * TILE
    xbt = jnp.pad(xb, ((0, n_pad - n), (0, 0))).T

    out = pl.pallas_call(
        _topk_body,
        grid=(n_pad // TILE,),
        in_specs=[
            pl.BlockSpec((Q, 16), lambda j: (0, 0)),
            pl.BlockSpec((16, TILE), lambda j: (0, j)),
        ],
        out_specs=pl.BlockSpec((K_SEL, TILE), lambda j: (0, j)),
        out_shape=jax.ShapeDtypeStruct((K_SEL, n_pad), jnp.int32),
    )(xq, xbt)
    return out[:, :n]


# speedup vs baseline: 19.7425x; 1.2081x over previous
import jax
import jax.numpy as jnp
from jax.experimental import pallas as pl

K_SEL = 21
Q = 1024
KRUN = 32
TILE = 256


def _stage_desc(v, i, dist):
    n, t = v.shape
    vr = v.reshape(n // (2 * dist), 2, dist, t)
    ir = i.reshape(n // (2 * dist), 2, dist, t)
    sel = vr[:, 0] >= vr[:, 1]
    fv = jnp.where(sel, vr[:, 0], vr[:, 1])
    fi = jnp.where(sel, ir[:, 0], ir[:, 1])
    sv = jnp.where(sel, vr[:, 1], vr[:, 0])
    si = jnp.where(sel, ir[:, 1], ir[:, 0])
    v = jnp.stack([fv, sv], axis=1).reshape(n, t)
    i = jnp.stack([fi, si], axis=1).reshape(n, t)
    return v, i


def _neg_bit(v, b):
    n, t = v.shape
    vr = v.reshape(n >> (b + 1), 2, 1 << b, t)
    return jnp.concatenate([vr[:, :1], -vr[:, 1:]], axis=1).reshape(n, t)


def _neg_xor(v, x, y):
    n, t = v.shape
    vr = v.reshape(n >> (x + 1), 2, 1 << (x - y - 1), 2, 1 << y, t)
    a = vr[:, 0]
    b = vr[:, 1]
    a = jnp.concatenate([a[:, :, :1], -a[:, :, 1:]], axis=2)
    b = jnp.concatenate([-b[:, :, :1], b[:, :, 1:]], axis=2)
    return jnp.stack([a, b], axis=1).reshape(n, t)


def _topk_body(xq_ref, xbt_ref, out_ref):
    s = jnp.dot(xq_ref[...], xbt_ref[...],
                preferred_element_type=jnp.float32)
    t = s.shape[-1]
    i = jax.lax.broadcasted_iota(jnp.int32, (Q, t), 0)

    v = _neg_xor(s, 6, 4)
    k = 2
    while k <= KRUN:
        d = k // 2
        while d >= 1:
            v, i = _stage_desc(v, i, d * KRUN)
            d //= 2
        if k < KRUN:
            kb = 5 + k.bit_length() - 1
            if 2 * k < KRUN:
                v = _neg_xor(v, kb + 1, kb)
            else:
                v = _neg_bit(v, kb)
        k *= 2

    sruns = KRUN
    while sruns > 1:
        n = v.shape[0]
        vr = v.reshape(n // sruns, 2, sruns // 2, t)
        ir = i.reshape(n // sruns, 2, sruns // 2, t)
        nb = -vr[:, 1]
        m = vr[:, 0] >= nb
        v = jnp.where(m, vr[:, 0], nb).reshape(n // 2, t)
        i = jnp.where(m, ir[:, 0], ir[:, 1]).reshape(n // 2, t)
        sruns //= 2
        if sruns > 1:
            v = _neg_bit(v, sruns.bit_length() - 2)
        d = KRUN // 2
        while d >= 1:
            v, i = _stage_desc(v, i, d * sruns)
            d //= 2

    for off in (0, 1):
        va = v[off:off + 30].reshape(15, 2, t)
        ia = i[off:off + 30].reshape(15, 2, t)
        beats = (va[:, 0] > va[:, 1]) | (
            (va[:, 0] == va[:, 1]) & (ia[:, 0] < ia[:, 1]))
        hv = jnp.where(beats, va[:, 0], va[:, 1])
        hi = jnp.where(beats, ia[:, 0], ia[:, 1])
        lv = jnp.where(beats, va[:, 1], va[:, 0])
        li = jnp.where(beats, ia[:, 1], ia[:, 0])
        vm = jnp.stack([hv, lv], axis=1).reshape(30, t)
        im = jnp.stack([hi, li], axis=1).reshape(30, t)
        vparts = [vm, v[off + 30:]] if off == 0 else [v[:off], vm, v[off + 30:]]
        iparts = [im, i[off + 30:]] if off == 0 else [i[:off], im, i[off + 30:]]
        v = jnp.concatenate(vparts, axis=0)
        i = jnp.concatenate(iparts, axis=0)

    out_ref[...] = i[:K_SEL]


def kernel(xq, xb):
    n = xb.shape[0]
    n_pad = ((n + TILE - 1) // TILE) * TILE
    xbt = jnp.pad(xb, ((0, n_pad - n), (0, 0))).T

    out = pl.pallas_call(
        _topk_body,
        grid=(n_pad // TILE,),
        in_specs=[
            pl.BlockSpec((Q, 16), lambda j: (0, 0)),
            pl.BlockSpec((16, TILE), lambda j: (0, j)),
        ],
        out_specs=pl.BlockSpec((K_SEL, TILE), lambda j: (0, j)),
        out_shape=jax.ShapeDtypeStruct((K_SEL, n_pad), jnp.int32),
    )(xq, xbt)
    return out[:, :n]


# speedup vs baseline: 22.4497x; 1.1371x over previous
import jax
import jax.numpy as jnp
from jax.experimental import pallas as pl

K_SEL = 21
Q = 1024
KRUN = 32
TILE = 512


def _stage_desc(v, i, dist):
    n, t = v.shape
    vr = v.reshape(n // (2 * dist), 2, dist, t)
    ir = i.reshape(n // (2 * dist), 2, dist, t)
    sel = vr[:, 0] >= vr[:, 1]
    fv = jnp.where(sel, vr[:, 0], vr[:, 1])
    fi = jnp.where(sel, ir[:, 0], ir[:, 1])
    sv = jnp.where(sel, vr[:, 1], vr[:, 0])
    si = jnp.where(sel, ir[:, 1], ir[:, 0])
    v = jnp.stack([fv, sv], axis=1).reshape(n, t)
    i = jnp.stack([fi, si], axis=1).reshape(n, t)
    return v, i


def _neg_bit(v, b):
    n, t = v.shape
    vr = v.reshape(n >> (b + 1), 2, 1 << b, t)
    return jnp.concatenate([vr[:, :1], -vr[:, 1:]], axis=1).reshape(n, t)


def _neg_xor(v, x, y):
    n, t = v.shape
    vr = v.reshape(n >> (x + 1), 2, 1 << (x - y - 1), 2, 1 << y, t)
    a = vr[:, 0]
    b = vr[:, 1]
    a = jnp.concatenate([a[:, :, :1], -a[:, :, 1:]], axis=2)
    b = jnp.concatenate([-b[:, :, :1], b[:, :, 1:]], axis=2)
    return jnp.stack([a, b], axis=1).reshape(n, t)


def _topk_body(xq_ref, xbt_ref, out_ref):
    s = jnp.dot(xq_ref[...], xbt_ref[...],
                preferred_element_type=jnp.float32)
    t = s.shape[-1]
    i = jax.lax.broadcasted_iota(jnp.int32, (Q, t), 0)

    v = _neg_xor(s, 6, 4)
    k = 2
    while k <= KRUN:
        d = k // 2
        while d >= 1:
            v, i = _stage_desc(v, i, d * KRUN)
            d //= 2
        if k < KRUN:
            kb = 5 + k.bit_length() - 1
            if 2 * k < KRUN:
                v = _neg_xor(v, kb + 1, kb)
            else:
                v = _neg_bit(v, kb)
        k *= 2

    sruns = KRUN
    while sruns > 1:
        n = v.shape[0]
        vr = v.reshape(n // sruns, 2, sruns // 2, t)
        ir = i.reshape(n // sruns, 2, sruns // 2, t)
        nb = -vr[:, 1]
        m = vr[:, 0] >= nb
        v = jnp.where(m, vr[:, 0], nb).reshape(n // 2, t)
        i = jnp.where(m, ir[:, 0], ir[:, 1]).reshape(n // 2, t)
        sruns //= 2
        if sruns > 1:
            v = _neg_bit(v, sruns.bit_length() - 2)
        d = KRUN // 2
        while d >= 1:
            v, i = _stage_desc(v, i, d * sruns)
            d //= 2

    for off in (0, 1):
        va = v[off:off + 30].reshape(15, 2, t)
        ia = i[off:off + 30].reshape(15, 2, t)
        beats = (va[:, 0] > va[:, 1]) | (
            (va[:, 0] == va[:, 1]) & (ia[:, 0] < ia[:, 1]))
        hv = jnp.where(beats, va[:, 0], va[:, 1])
        hi = jnp.where(beats, ia[:, 0], ia[:, 1])
        lv = jnp.where(beats, va[:, 1], va[:, 0])
        li = jnp.where(beats, ia[:, 1], ia[:, 0])
        vm = jnp.stack([hv, lv], axis=1).reshape(30, t)
        im = jnp.stack([hi, li], axis=1).reshape(30, t)
        vparts = [vm, v[off + 30:]] if off == 0 else [v[:off], vm, v[off + 30:]]
        iparts = [im, i[off + 30:]] if off == 0 else [i[:off], im, i[off + 30:]]
        v = jnp.concatenate(vparts, axis=0)
        i = jnp.concatenate(iparts, axis=0)

    out_ref[...] = i[:K_SEL]


def kernel(xq, xb):
    n = xb.shape[0]
    n_pad = ((n + TILE - 1) // TILE) * TILE
    xbt = jnp.pad(xb, ((0, n_pad - n), (0, 0))).T

    out = pl.pallas_call(
        _topk_body,
        grid=(n_pad // TILE,),
        in_specs=[
            pl.BlockSpec((Q, 16), lambda j: (0, 0)),
            pl.BlockSpec((16, TILE), lambda j: (0, j)),
        ],
        out_specs=pl.BlockSpec((K_SEL, TILE), lambda j: (0, j)),
        out_shape=jax.ShapeDtypeStruct((K_SEL, n_pad), jnp.int32),
    )(xq, xbt)
    return out[:, :n]


# speedup vs baseline: 23.4945x; 1.0465x over previous
import jax
import jax.numpy as jnp
from jax.experimental import pallas as pl

K_SEL = 21
Q = 1024
KRUN = 32
TILE = 1024


def _stage_desc(v, i, dist):
    n, t = v.shape
    vr = v.reshape(n // (2 * dist), 2, dist, t)
    ir = i.reshape(n // (2 * dist), 2, dist, t)
    sel = vr[:, 0] >= vr[:, 1]
    fv = jnp.where(sel, vr[:, 0], vr[:, 1])
    fi = jnp.where(sel, ir[:, 0], ir[:, 1])
    sv = jnp.where(sel, vr[:, 1], vr[:, 0])
    si = jnp.where(sel, ir[:, 1], ir[:, 0])
    v = jnp.stack([fv, sv], axis=1).reshape(n, t)
    i = jnp.stack([fi, si], axis=1).reshape(n, t)
    return v, i


def _neg_bit(v, b):
    n, t = v.shape
    vr = v.reshape(n >> (b + 1), 2, 1 << b, t)
    return jnp.concatenate([vr[:, :1], -vr[:, 1:]], axis=1).reshape(n, t)


def _neg_xor(v, x, y):
    n, t = v.shape
    vr = v.reshape(n >> (x + 1), 2, 1 << (x - y - 1), 2, 1 << y, t)
    a = vr[:, 0]
    b = vr[:, 1]
    a = jnp.concatenate([a[:, :, :1], -a[:, :, 1:]], axis=2)
    b = jnp.concatenate([-b[:, :, :1], b[:, :, 1:]], axis=2)
    return jnp.stack([a, b], axis=1).reshape(n, t)


def _topk_body(xq_ref, xbt_ref, out_ref):
    s = jnp.dot(xq_ref[...], xbt_ref[...],
                preferred_element_type=jnp.float32)
    t = s.shape[-1]
    i = jax.lax.broadcasted_iota(jnp.int32, (Q, t), 0)

    v = _neg_xor(s, 6, 4)
    k = 2
    while k <= KRUN:
        d = k // 2
        while d >= 1:
            v, i = _stage_desc(v, i, d * KRUN)
            d //= 2
        if k < KRUN:
            kb = 5 + k.bit_length() - 1
            if 2 * k < KRUN:
                v = _neg_xor(v, kb + 1, kb)
            else:
                v = _neg_bit(v, kb)
        k *= 2

    sruns = KRUN
    while sruns > 1:
        n = v.shape[0]
        vr = v.reshape(n // sruns, 2, sruns // 2, t)
        ir = i.reshape(n // sruns, 2, sruns // 2, t)
        nb = -vr[:, 1]
        m = vr[:, 0] >= nb
        v = jnp.where(m, vr[:, 0], nb).reshape(n // 2, t)
        i = jnp.where(m, ir[:, 0], ir[:, 1]).reshape(n // 2, t)
        sruns //= 2
        if sruns > 1:
            v = _neg_bit(v, sruns.bit_length() - 2)
        d = KRUN // 2
        while d >= 1:
            v, i = _stage_desc(v, i, d * sruns)
            d //= 2

    for off in (0, 1):
        va = v[off:off + 30].reshape(15, 2, t)
        ia = i[off:off + 30].reshape(15, 2, t)
        beats = (va[:, 0] > va[:, 1]) | (
            (va[:, 0] == va[:, 1]) & (ia[:, 0] < ia[:, 1]))
        hv = jnp.where(beats, va[:, 0], va[:, 1])
        hi = jnp.where(beats, ia[:, 0], ia[:, 1])
        lv = jnp.where(beats, va[:, 1], va[:, 0])
        li = jnp.where(beats, ia[:, 1], ia[:, 0])
        vm = jnp.stack([hv, lv], axis=1).reshape(30, t)
        im = jnp.stack([hi, li], axis=1).reshape(30, t)
        vparts = [vm, v[off + 30:]] if off == 0 else [v[:off], vm, v[off + 30:]]
        iparts = [im, i[off + 30:]] if off == 0 else [i[:off], im, i[off + 30:]]
        v = jnp.concatenate(vparts, axis=0)
        i = jnp.concatenate(iparts, axis=0)

    out_ref[...] = i[:K_SEL]


def kernel(xq, xb):
    n = xb.shape[0]
    n_pad = ((n + TILE - 1) // TILE) * TILE
    xbt = jnp.pad(xb, ((0, n_pad - n), (0, 0))).T

    out = pl.pallas_call(
        _topk_body,
        grid=(n_pad // TILE,),
        in_specs=[
            pl.BlockSpec((Q, 16), lambda j: (0, 0)),
            pl.BlockSpec((16, TILE), lambda j: (0, j)),
        ],
        out_specs=pl.BlockSpec((K_SEL, TILE), lambda j: (0, j)),
        out_shape=jax.ShapeDtypeStruct((K_SEL, n_pad), jnp.int32),
    )(xq, xbt)
    return out[:, :n]
